# Initial kernel scaffold; baseline (speedup 1.0000x reference)
import jax
import jax.numpy as jnp
from jax.experimental import pallas as pl


def kernel(feature, edge_index, W):
    raise NotImplementedError("write your pallas kernel here")


# speedup vs baseline: 7.5896x; 7.5896x over previous
import functools

import jax
import jax.numpy as jnp
from jax import lax
from jax.experimental import pallas as pl
from jax.experimental.pallas import tpu as pltpu
from jax.experimental.pallas import tpu_sc as plsc

N_NODES = 10000
N_EDGES = 320000
D = 128

NC = 2
NS = 16
NW = NC * NS

CHUNK = 80
E_PER_W = N_EDGES // NW
N_CHUNKS = E_PER_W // CHUNK
N_PAD = 10240
ROWS_PER_TILE = N_PAD // NS

_sc_mesh = plsc.VectorSubcoreMesh(core_axis_name="c", subcore_axis_name="s")


@functools.partial(
    pl.kernel,
    out_type=jax.ShapeDtypeStruct((NC, N_PAD, D), jnp.float32),
    mesh=_sc_mesh,
    scratch_types=[
        pltpu.VMEM_SHARED((N_PAD, D), jnp.float32),
        pltpu.VMEM((N_CHUNKS, CHUNK), jnp.int32),
        pltpu.VMEM((N_CHUNKS, CHUNK), jnp.int32),
        pltpu.VMEM((CHUNK, D), jnp.float32),
        pltpu.SemaphoreType.DMA,
    ],
)
def _sc_scatter(feature_hbm, src_hbm, dst_hbm, zeros_hbm, out_hbm,
                acc, src_v, dst_v, rows_v, sem):
    cid = lax.axis_index("c")
    sid = lax.axis_index("s")
    wid = sid * NC + cid

    row0 = sid * ROWS_PER_TILE
    pltpu.sync_copy(zeros_hbm.at[pl.ds(row0, ROWS_PER_TILE)],
                    acc.at[pl.ds(row0, ROWS_PER_TILE)])

    pltpu.sync_copy(src_hbm.at[wid], src_v)
    pltpu.sync_copy(dst_hbm.at[wid], dst_v)
    plsc.subcore_barrier()

    def step(k, _):
        pltpu.async_copy(feature_hbm.at[src_v.at[k]], rows_v, sem).wait()
        pltpu.sync_copy(rows_v, acc.at[dst_v.at[k]], add=True)
        return _

    lax.fori_loop(0, N_CHUNKS, step, None)

    plsc.subcore_barrier()
    pltpu.sync_copy(acc.at[pl.ds(row0, ROWS_PER_TILE)],
                    out_hbm.at[cid, pl.ds(row0, ROWS_PER_TILE)])


ROWS_TC = 1000


def _tc_body(p_ref, w_ref, o_ref):
    s = p_ref[0] + p_ref[1]
    o_ref[...] = jnp.maximum(
        jnp.dot(s, w_ref[...], preferred_element_type=jnp.float32), 0.0)


_tc_matmul = pl.pallas_call(
    _tc_body,
    grid=(N_NODES // ROWS_TC,),
    in_specs=[
        pl.BlockSpec((NC, ROWS_TC, D), lambda i: (0, i, 0)),
        pl.BlockSpec((D, D), lambda i: (0, 0)),
    ],
    out_specs=pl.BlockSpec((ROWS_TC, D), lambda i: (i, 0)),
    out_shape=jax.ShapeDtypeStruct((N_NODES, D), jnp.float32),
)


def kernel(feature, edge_index, W):
    src = edge_index[0].astype(jnp.int32).reshape(NW, N_CHUNKS, CHUNK)
    dst = edge_index[1].astype(jnp.int32).reshape(NW, N_CHUNKS, CHUNK)
    zeros = jnp.zeros((N_PAD, D), jnp.float32)
    partial = _sc_scatter(feature, src, dst, zeros)
    return _tc_matmul(partial, W.T)
